# Initial kernel scaffold; baseline (speedup 1.0000x reference)
import jax
import jax.numpy as jnp
from jax.experimental import pallas as pl


def kernel(x, batch, single_edge_index, single_edge_attr, double_edge_index, double_edge_attr, triple_edge_index, triple_edge_attr, aromatic_edge_index, aromatic_edge_attr, x_embedding1, x_embedding2, edge_embedding1, edge_embedding2, l0_w1, l0_w2, l1_w1, gcn_W, gcn_b, lin_W, lin_b):
    raise NotImplementedError("write your pallas kernel here")


# speedup vs baseline: 3.7624x; 3.7624x over previous
import functools

import jax
import jax.numpy as jnp
from jax import lax
from jax.experimental import pallas as pl
from jax.experimental.pallas import tpu as pltpu
from jax.experimental.pallas import tpu_sc as plsc

N = 1024
EMB = 128
WOUT = 128
NCH = 2
NG = 64
NSUB = 16

E = 9216
REL_OFF = (0, 4096, 6144, 7168)
REL_N = (4096, 2048, 1024, 2048)
MAT = N * N
MSL = MAT // NSUB

f32 = jnp.float32
bf16 = jnp.bfloat16
i32 = jnp.int32
_HI = jax.lax.Precision.HIGHEST


def _build_relation(r, sid, rows_h, cols_h, a0_h, a1_h, zeros_h, rel_h,
                    r_v, c_v, a0_v, a1_v, idx_v, upd_v, ee1_v, ee2_v, matS):
    npersub = REL_N[r] // NSUB
    nchunk = npersub // 16
    eb = pl.multiple_of(REL_OFF[r] + sid * npersub, 8)
    pltpu.sync_copy(rows_h.at[pl.ds(eb, npersub)], r_v.at[pl.ds(0, npersub)])
    pltpu.sync_copy(cols_h.at[pl.ds(eb, npersub)], c_v.at[pl.ds(0, npersub)])
    pltpu.sync_copy(a0_h.at[pl.ds(eb, npersub)], a0_v.at[pl.ds(0, npersub)])
    pltpu.sync_copy(a1_h.at[pl.ds(eb, npersub)], a1_v.at[pl.ds(0, npersub)])
    zb = pl.multiple_of(sid * MSL, MSL)
    pltpu.sync_copy(zeros_h.at[pl.ds(zb, MSL)], matS.at[pl.ds(zb, MSL)])
    for q in range(nchunk):
        sl = pl.ds(q * 16, 16)
        val = (plsc.load_gather(ee1_v, [a0_v[sl]]) +
               plsc.load_gather(ee2_v, [a1_v[sl]]))
        k, off = divmod(q * 16, 128)
        idx_v[k, pl.ds(off, 16)] = r_v[sl] * N + c_v[sl]
        upd_v[k, pl.ds(off, 16)] = val
    for q in range(nchunk, ((nchunk + 7) // 8) * 8):
        k, off = divmod(q * 16, 128)
        idx_v[k, pl.ds(off, 16)] = jnp.zeros((16,), i32)
        upd_v[k, pl.ds(off, 16)] = jnp.zeros((16,), f32)
    plsc.subcore_barrier()
    for k in range((nchunk + 7) // 8):
        pltpu.sync_copy(upd_v.at[k], matS.at[idx_v.at[k]], add=True)
    plsc.subcore_barrier()
    ob = pl.multiple_of(r * MAT + zb, MSL)
    pltpu.sync_copy(matS.at[pl.ds(zb, MSL)], rel_h.at[pl.ds(ob, MSL)])
    plsc.subcore_barrier()


def _sc_body(rows_h, cols_h, a0_h, a1_h, ee1_h, ee2_h, x0_h, x1_h,
             emb1_h, emb2_h, zeros_h,
             rel_h, h_out_h,
             r_v, c_v, a0_v, a1_v, idx_v, upd_v, ee1_v, ee2_v,
             xi_v, g1_v, g2_v, matS, sem):
    cid = lax.axis_index("c")
    sid = lax.axis_index("s")

    wid = sid * 2 + cid
    nb = pl.multiple_of(wid * 32, 32)
    pltpu.sync_copy(x0_h.at[pl.ds(nb, 32)], xi_v)
    pltpu.async_copy(emb1_h.at[xi_v], g1_v, sem).wait()
    pltpu.sync_copy(x1_h.at[pl.ds(nb, 32)], xi_v)
    pltpu.async_copy(emb2_h.at[xi_v], g2_v, sem).wait()
    for j in range(32):
        for q in range(8):
            sl = pl.ds(q * 16, 16)
            g1_v[j, sl] = g1_v[j, sl] + g2_v[j, sl]
    pltpu.sync_copy(g1_v, h_out_h.at[pl.ds(nb, 32)])

    pltpu.sync_copy(ee1_h, ee1_v)
    pltpu.sync_copy(ee2_h, ee2_v)

    args = (rows_h, cols_h, a0_h, a1_h, zeros_h, rel_h,
            r_v, c_v, a0_v, a1_v, idx_v, upd_v, ee1_v, ee2_v, matS)

    @pl.when(cid == 0)
    def _():
        _build_relation(0, sid, *args)
        _build_relation(2, sid, *args)

    @pl.when(cid == 1)
    def _():
        _build_relation(1, sid, *args)
        _build_relation(3, sid, *args)


@jax.jit
def _sc_scatter(rows_p, cols_p, a0_p, a1_p, ee1p, ee2p, x0, x1,
                emb1, emb2, zeros_flat):
    mesh = plsc.VectorSubcoreMesh(core_axis_name="c", subcore_axis_name="s")
    return pl.kernel(
        _sc_body,
        out_type=[jax.ShapeDtypeStruct((4 * MAT,), f32),
                  jax.ShapeDtypeStruct((N, EMB), f32)],
        mesh=mesh,
        scratch_types=[
            pltpu.VMEM((256,), i32),
            pltpu.VMEM((256,), i32),
            pltpu.VMEM((256,), i32),
            pltpu.VMEM((256,), i32),
            pltpu.VMEM((2, 128), i32),
            pltpu.VMEM((2, 128), f32),
            pltpu.VMEM((16,), f32),
            pltpu.VMEM((16,), f32),
            pltpu.VMEM((32,), i32),
            pltpu.VMEM((32, 128), f32),
            pltpu.VMEM((32, 128), f32),
            pltpu.VMEM_SHARED((MAT,), f32),
            pltpu.SemaphoreType.DMA,
        ],
        compiler_params=pltpu.CompilerParams(needs_layout_passes=False),
    )(rows_p, cols_p, a0_p, a1_p, ee1p, ee2p, x0, x1, emb1, emb2, zeros_flat)


def _tc_body(fb_ref, rel_ref, h_ref, gcnW_ref, gcnb_ref, batch_ref,
             linW_ref, linb_ref, out_ref):
    c = pl.program_id(0)
    dotb = functools.partial(lax.dot, preferred_element_type=f32)

    iota_r = lax.broadcasted_iota(i32, (N, N), 0)
    iota_c = lax.broadcasted_iota(i32, (N, N), 1)
    diag = iota_r == iota_c

    def combine(m):
        acc = fb_ref[c, m, 0] * rel_ref[0].astype(bf16).astype(f32)
        for j in range(1, 4):
            acc = acc + fb_ref[c, m, j] * rel_ref[j].astype(bf16).astype(f32)
        acc = jnp.where(diag, acc + fb_ref[c, m, 4], acc)
        return acc.astype(bf16)

    Am = combine(0)
    Bm = combine(1)
    Cm = combine(2)

    hWb = dotb(h_ref[...].astype(bf16), gcnW_ref[...].astype(bf16))
    H0 = dotb(Am, Bm)
    deg0 = jnp.sum(H0, axis=1, keepdims=True)
    inv0 = jnp.where(deg0 != 0.0, 1.0 / deg0, 0.0)
    H1 = dotb((inv0 * H0).astype(bf16), Cm)
    deg1 = jnp.sum(H1, axis=1, keepdims=True)
    inv1 = jnp.where(deg1 != 0.0, 1.0 / deg1, 0.0)
    Xc = jnp.maximum(dotb((inv1 * H1).astype(bf16), hWb.astype(bf16))
                     + gcnb_ref[...], 0.0)

    onehot = (lax.broadcasted_iota(i32, (NG, N), 0)
              == batch_ref[...]).astype(f32)
    counts = jnp.maximum(jnp.sum(onehot, axis=1, keepdims=True), 1.0)
    yc = lax.dot(onehot, Xc, precision=_HI,
                 preferred_element_type=f32) / counts
    contrib = dotb(yc.astype(bf16), linW_ref[0].astype(bf16))

    @pl.when(c == 0)
    def _():
        out_ref[...] = contrib + linb_ref[...]

    @pl.when(c != 0)
    def _():
        out_ref[...] = out_ref[...] + contrib


@jax.jit
def _tc_replica(fb, relstack, h, gcn_W, gcn_b2, batch2, lin_W3, lin_b2):
    return pl.pallas_call(
        _tc_body,
        grid=(NCH,),
        in_specs=[
            pl.BlockSpec(memory_space=pltpu.MemorySpace.SMEM),
            pl.BlockSpec((4, N, N), lambda c: (0, 0, 0)),
            pl.BlockSpec((N, EMB), lambda c: (0, 0)),
            pl.BlockSpec((EMB, WOUT), lambda c: (0, 0)),
            pl.BlockSpec((1, WOUT), lambda c: (0, 0)),
            pl.BlockSpec((1, N), lambda c: (0, 0)),
            pl.BlockSpec((1, WOUT, WOUT), lambda c: (c, 0, 0)),
            pl.BlockSpec((1, WOUT), lambda c: (0, 0)),
        ],
        out_specs=pl.BlockSpec((NG, WOUT), lambda c: (0, 0)),
        out_shape=jax.ShapeDtypeStruct((NG, WOUT), f32),
        compiler_params=pltpu.CompilerParams(
            dimension_semantics=("arbitrary",),
            vmem_limit_bytes=100 * 1024 * 1024),
    )(fb, relstack, h, gcn_W, gcn_b2, batch2, lin_W3, lin_b2)


def kernel(x, batch, single_edge_index, single_edge_attr, double_edge_index,
           double_edge_attr, triple_edge_index, triple_edge_attr,
           aromatic_edge_index, aromatic_edge_attr, x_embedding1, x_embedding2,
           edge_embedding1, edge_embedding2, l0_w1, l0_w2, l1_w1, gcn_W,
           gcn_b, lin_W, lin_b):
    idxs = (single_edge_index, double_edge_index, triple_edge_index,
            aromatic_edge_index)
    attrs = (single_edge_attr, double_edge_attr, triple_edge_attr,
             aromatic_edge_attr)

    rows_p = jnp.concatenate([i[0] for i in idxs]).astype(i32)
    cols_p = jnp.concatenate([i[1] for i in idxs]).astype(i32)
    a0_p = jnp.concatenate([a[:, 0] for a in attrs]).astype(i32)
    a1_p = jnp.concatenate([a[:, 1] for a in attrs]).astype(i32)

    ee1p = jnp.pad(edge_embedding1[:, 0].astype(f32), (0, 11))
    ee2p = jnp.pad(edge_embedding2[:, 0].astype(f32), (0, 13))

    x0 = x[:, 0].astype(i32)
    x1 = x[:, 1].astype(i32)
    zeros_flat = jnp.zeros((MAT,), f32)

    relflat, h = _sc_scatter(rows_p, cols_p, a0_p, a1_p, ee1p, ee2p,
                             x0, x1, x_embedding1.astype(f32),
                             x_embedding2.astype(f32), zeros_flat)
    relstack = relflat.reshape(4, N, N)

    fb = jnp.stack([jax.nn.softmax(w.astype(f32), axis=1)
                    for w in (l0_w1, l0_w2, l1_w1)])
    fb = fb.astype(bf16).astype(f32).transpose(1, 0, 2)

    return _tc_replica(fb, relstack, h, gcn_W.astype(f32),
                       gcn_b.astype(f32).reshape(1, WOUT),
                       batch.astype(i32).reshape(1, N),
                       lin_W.astype(f32).reshape(NCH, WOUT, WOUT),
                       lin_b.astype(f32).reshape(1, WOUT))


# speedup vs baseline: 4.4153x; 1.1735x over previous
import functools

import jax
import jax.numpy as jnp
from jax import lax
from jax.experimental import pallas as pl
from jax.experimental.pallas import tpu as pltpu
from jax.experimental.pallas import tpu_sc as plsc

N = 1024
EMB = 128
WOUT = 128
NCH = 2
NG = 64
NSUB = 16

E = 9216
REL_OFF = (0, 4096, 6144, 7168)
REL_N = (4096, 2048, 1024, 2048)
MAT = N * N
MSL = MAT // NSUB
ZB = 16384

f32 = jnp.float32
bf16 = jnp.bfloat16
i32 = jnp.int32
_HI = jax.lax.Precision.HIGHEST


def _build_relation(r, sid, bufs, zbuf, rel_h,
                    idx_v, upd_v, ee1_v, ee2_v, matS, semz):
    r_v, c_v, a0_v, a1_v = bufs
    npersub = REL_N[r] // NSUB
    nchunk = npersub // 16
    zb = pl.multiple_of(sid * MSL, MSL)
    zd = [pltpu.async_copy(zbuf, matS.at[pl.ds(zb + q * ZB, ZB)], semz)
          for q in range(MSL // ZB)]
    for q in range(nchunk):
        sl = pl.ds(q * 16, 16)
        val = (plsc.load_gather(ee1_v, [a0_v[sl]]) +
               plsc.load_gather(ee2_v, [a1_v[sl]]))
        k, off = divmod(q * 16, 128)
        idx_v[k, pl.ds(off, 16)] = r_v[sl] * N + c_v[sl]
        upd_v[k, pl.ds(off, 16)] = val
    for q in range(nchunk, ((nchunk + 7) // 8) * 8):
        k, off = divmod(q * 16, 128)
        idx_v[k, pl.ds(off, 16)] = jnp.zeros((16,), i32)
        upd_v[k, pl.ds(off, 16)] = jnp.zeros((16,), f32)
    for d in zd:
        d.wait()
    plsc.subcore_barrier()
    for k in range((nchunk + 7) // 8):
        pltpu.sync_copy(upd_v.at[k], matS.at[idx_v.at[k]], add=True)
    plsc.subcore_barrier()
    ob = pl.multiple_of(r * MAT + zb, MSL)
    pltpu.sync_copy(matS.at[pl.ds(zb, MSL)], rel_h.at[pl.ds(ob, MSL)])
    plsc.subcore_barrier()


def _sc_body(rows_h, cols_h, a0_h, a1_h, ee1_h, ee2_h, x0_h, x1_h,
             emb1_h, emb2_h, zeros_h,
             rel_h, h_out_h,
             rA_v, cA_v, a0A_v, a1A_v, rB_v, cB_v, a0B_v, a1B_v,
             idx_v, upd_v, ee1_v, ee2_v, zbuf,
             xi0_v, xi1_v, g1_v, g2_v, matS, semi, semg, semz):
    cid = lax.axis_index("c")
    sid = lax.axis_index("s")

    wid = sid * 2 + cid
    nb = pl.multiple_of(wid * 32, 32)
    descs = [
        pltpu.async_copy(zeros_h, zbuf, semi),
        pltpu.async_copy(x0_h.at[pl.ds(nb, 32)], xi0_v, semi),
        pltpu.async_copy(x1_h.at[pl.ds(nb, 32)], xi1_v, semi),
        pltpu.async_copy(ee1_h, ee1_v, semi),
        pltpu.async_copy(ee2_h, ee2_v, semi),
    ]

    def edge_loads(r, bufs):
        npersub = REL_N[r] // NSUB
        eb = pl.multiple_of(REL_OFF[r] + sid * npersub, 8)
        srcs = (rows_h, cols_h, a0_h, a1_h)
        return [pltpu.async_copy(s.at[pl.ds(eb, npersub)],
                                 b.at[pl.ds(0, npersub)], semi)
                for s, b in zip(srcs, bufs)]

    bufsA = (rA_v, cA_v, a0A_v, a1A_v)
    bufsB = (rB_v, cB_v, a0B_v, a1B_v)

    @pl.when(cid == 0)
    def _():
        for d in edge_loads(0, bufsA) + edge_loads(2, bufsB):
            d.wait()

    @pl.when(cid == 1)
    def _():
        for d in edge_loads(1, bufsA) + edge_loads(3, bufsB):
            d.wait()

    for d in descs:
        d.wait()

    dg1 = pltpu.async_copy(emb1_h.at[xi0_v], g1_v, semg)
    dg2 = pltpu.async_copy(emb2_h.at[xi1_v], g2_v, semg)
    dg1.wait()
    dg2.wait()
    for j in range(32):
        for q in range(8):
            sl = pl.ds(q * 16, 16)
            g1_v[j, sl] = g1_v[j, sl] + g2_v[j, sl]
    pltpu.sync_copy(g1_v, h_out_h.at[pl.ds(nb, 32)])

    @pl.when(cid == 0)
    def _():
        _build_relation(0, sid, bufsA, zbuf, rel_h,
                        idx_v, upd_v, ee1_v, ee2_v, matS, semz)
        _build_relation(2, sid, bufsB, zbuf, rel_h,
                        idx_v, upd_v, ee1_v, ee2_v, matS, semz)

    @pl.when(cid == 1)
    def _():
        _build_relation(1, sid, bufsA, zbuf, rel_h,
                        idx_v, upd_v, ee1_v, ee2_v, matS, semz)
        _build_relation(3, sid, bufsB, zbuf, rel_h,
                        idx_v, upd_v, ee1_v, ee2_v, matS, semz)


@jax.jit
def _sc_scatter(rows_p, cols_p, a0_p, a1_p, ee1p, ee2p, x0, x1,
                emb1, emb2, zeros16k):
    mesh = plsc.VectorSubcoreMesh(core_axis_name="c", subcore_axis_name="s")
    return pl.kernel(
        _sc_body,
        out_type=[jax.ShapeDtypeStruct((4 * MAT,), f32),
                  jax.ShapeDtypeStruct((N, EMB), f32)],
        mesh=mesh,
        scratch_types=[
            pltpu.VMEM((256,), i32),
            pltpu.VMEM((256,), i32),
            pltpu.VMEM((256,), i32),
            pltpu.VMEM((256,), i32),
            pltpu.VMEM((128,), i32),
            pltpu.VMEM((128,), i32),
            pltpu.VMEM((128,), i32),
            pltpu.VMEM((128,), i32),
            pltpu.VMEM((2, 128), i32),
            pltpu.VMEM((2, 128), f32),
            pltpu.VMEM((16,), f32),
            pltpu.VMEM((16,), f32),
            pltpu.VMEM((ZB,), f32),
            pltpu.VMEM((32,), i32),
            pltpu.VMEM((32,), i32),
            pltpu.VMEM((32, 128), f32),
            pltpu.VMEM((32, 128), f32),
            pltpu.VMEM_SHARED((MAT,), f32),
            pltpu.SemaphoreType.DMA,
            pltpu.SemaphoreType.DMA,
            pltpu.SemaphoreType.DMA,
        ],
        compiler_params=pltpu.CompilerParams(needs_layout_passes=False),
    )(rows_p, cols_p, a0_p, a1_p, ee1p, ee2p, x0, x1, emb1, emb2, zeros16k)


def _tc_body(fb_ref, rel_ref, h_ref, gcnW_ref, gcnb_ref, batch_ref,
             linW_ref, linb_ref, out_ref):
    c = pl.program_id(0)
    dotb = functools.partial(lax.dot, preferred_element_type=f32)

    iota_r = lax.broadcasted_iota(i32, (N, N), 0)
    iota_c = lax.broadcasted_iota(i32, (N, N), 1)
    diag = iota_r == iota_c

    def combine(m):
        acc = fb_ref[c, m, 0] * rel_ref[0].astype(f32)
        for j in range(1, 4):
            acc = acc + fb_ref[c, m, j] * rel_ref[j].astype(f32)
        acc = jnp.where(diag, acc + fb_ref[c, m, 4], acc)
        return acc.astype(bf16)

    Am = combine(0)
    Bm = combine(1)
    Cm = combine(2)

    hWb = dotb(h_ref[...].astype(bf16), gcnW_ref[...].astype(bf16))
    H0 = dotb(Am, Bm)
    deg0 = jnp.sum(H0, axis=1, keepdims=True)
    inv0 = jnp.where(deg0 != 0.0, 1.0 / deg0, 0.0)
    H1 = dotb((inv0 * H0).astype(bf16), Cm)
    deg1 = jnp.sum(H1, axis=1, keepdims=True)
    inv1 = jnp.where(deg1 != 0.0, 1.0 / deg1, 0.0)
    Xc = jnp.maximum(dotb((inv1 * H1).astype(bf16), hWb.astype(bf16))
                     + gcnb_ref[...], 0.0)

    onehot = (lax.broadcasted_iota(i32, (NG, N), 0)
              == batch_ref[...]).astype(f32)
    counts = jnp.maximum(jnp.sum(onehot, axis=1, keepdims=True), 1.0)
    yc = lax.dot(onehot, Xc, precision=_HI,
                 preferred_element_type=f32) / counts
    contrib = dotb(yc.astype(bf16), linW_ref[0].astype(bf16))

    @pl.when(c == 0)
    def _():
        out_ref[...] = contrib + linb_ref[...]

    @pl.when(c != 0)
    def _():
        out_ref[...] = out_ref[...] + contrib


@jax.jit
def _tc_replica(fb, relstack, h, gcn_W, gcn_b2, batch2, lin_W3, lin_b2):
    return pl.pallas_call(
        _tc_body,
        grid=(NCH,),
        in_specs=[
            pl.BlockSpec(memory_space=pltpu.MemorySpace.SMEM),
            pl.BlockSpec((4, N, N), lambda c: (0, 0, 0)),
            pl.BlockSpec((N, EMB), lambda c: (0, 0)),
            pl.BlockSpec((EMB, WOUT), lambda c: (0, 0)),
            pl.BlockSpec((1, WOUT), lambda c: (0, 0)),
            pl.BlockSpec((1, N), lambda c: (0, 0)),
            pl.BlockSpec((1, WOUT, WOUT), lambda c: (c, 0, 0)),
            pl.BlockSpec((1, WOUT), lambda c: (0, 0)),
        ],
        out_specs=pl.BlockSpec((NG, WOUT), lambda c: (0, 0)),
        out_shape=jax.ShapeDtypeStruct((NG, WOUT), f32),
        compiler_params=pltpu.CompilerParams(
            dimension_semantics=("arbitrary",),
            vmem_limit_bytes=100 * 1024 * 1024),
    )(fb, relstack, h, gcn_W, gcn_b2, batch2, lin_W3, lin_b2)


def kernel(x, batch, single_edge_index, single_edge_attr, double_edge_index,
           double_edge_attr, triple_edge_index, triple_edge_attr,
           aromatic_edge_index, aromatic_edge_attr, x_embedding1, x_embedding2,
           edge_embedding1, edge_embedding2, l0_w1, l0_w2, l1_w1, gcn_W,
           gcn_b, lin_W, lin_b):
    idxs = (single_edge_index, double_edge_index, triple_edge_index,
            aromatic_edge_index)
    attrs = (single_edge_attr, double_edge_attr, triple_edge_attr,
             aromatic_edge_attr)

    rows_p = jnp.concatenate([i[0] for i in idxs]).astype(i32)
    cols_p = jnp.concatenate([i[1] for i in idxs]).astype(i32)
    a0_p = jnp.concatenate([a[:, 0] for a in attrs]).astype(i32)
    a1_p = jnp.concatenate([a[:, 1] for a in attrs]).astype(i32)

    ee1p = jnp.pad(edge_embedding1[:, 0].astype(f32), (0, 11))
    ee2p = jnp.pad(edge_embedding2[:, 0].astype(f32), (0, 13))

    x0 = x[:, 0].astype(i32)
    x1 = x[:, 1].astype(i32)
    zeros16k = jnp.zeros((ZB,), f32)

    relflat, h = _sc_scatter(rows_p, cols_p, a0_p, a1_p, ee1p, ee2p,
                             x0, x1, x_embedding1.astype(f32),
                             x_embedding2.astype(f32), zeros16k)
    relstack = relflat.reshape(4, N, N).astype(bf16)

    fb = jnp.stack([jax.nn.softmax(w.astype(f32), axis=1)
                    for w in (l0_w1, l0_w2, l1_w1)])
    fb = fb.astype(bf16).astype(f32).transpose(1, 0, 2)

    return _tc_replica(fb, relstack, h, gcn_W.astype(f32),
                       gcn_b.astype(f32).reshape(1, WOUT),
                       batch.astype(i32).reshape(1, N),
                       lin_W.astype(f32).reshape(NCH, WOUT, WOUT),
                       lin_b.astype(f32).reshape(1, WOUT))


# speedup vs baseline: 4.4639x; 1.0110x over previous
import functools

import jax
import jax.numpy as jnp
from jax import lax
from jax.experimental import pallas as pl
from jax.experimental.pallas import tpu as pltpu
from jax.experimental.pallas import tpu_sc as plsc

N = 1024
EMB = 128
WOUT = 128
NCH = 2
NG = 64
NSUB = 16

E = 9216
REL_OFF = (0, 4096, 6144, 7168)
REL_N = (4096, 2048, 1024, 2048)
MAT = N * N
MSL = MAT // NSUB
ZB = 16384

f32 = jnp.float32
bf16 = jnp.bfloat16
i32 = jnp.int32
_HI = jax.lax.Precision.HIGHEST


def _build_relation(r, sid, bufs, zbuf, rel_h,
                    idx_v, upd_v, ee1_v, ee2_v, matS, semz, prev):
    r_v, c_v, a0_v, a1_v = bufs
    npersub = REL_N[r] // NSUB
    nchunk = npersub // 16
    nrow = (nchunk + 7) // 8
    zb = pl.multiple_of(sid * MSL, MSL)
    if prev is None:
        zd = [pltpu.async_copy(zbuf, matS.at[pl.ds(zb + q * ZB, ZB)], semz)
              for q in range(MSL // ZB)]
    else:
        pidx_v, pupd_v, pnrow = prev
        for q in range(pnrow * 8):
            k, off = divmod(q * 16, 128)
            sl = pl.ds(off, 16)
            pupd_v[k, sl] = -pupd_v[k, sl]
        for k in range(pnrow):
            pltpu.sync_copy(pupd_v.at[k], matS.at[pidx_v.at[k]], add=True)
    for q in range(nchunk):
        sl = pl.ds(q * 16, 16)
        val = (plsc.load_gather(ee1_v, [a0_v[sl]]) +
               plsc.load_gather(ee2_v, [a1_v[sl]]))
        k, off = divmod(q * 16, 128)
        idx_v[k, pl.ds(off, 16)] = r_v[sl] * N + c_v[sl]
        upd_v[k, pl.ds(off, 16)] = val
    for q in range(nchunk, nrow * 8):
        k, off = divmod(q * 16, 128)
        idx_v[k, pl.ds(off, 16)] = jnp.zeros((16,), i32)
        upd_v[k, pl.ds(off, 16)] = jnp.zeros((16,), f32)
    if prev is None:
        for d in zd:
            d.wait()
    plsc.subcore_barrier()
    for k in range(nrow):
        pltpu.sync_copy(upd_v.at[k], matS.at[idx_v.at[k]], add=True)
    plsc.subcore_barrier()
    ob = pl.multiple_of(r * MAT + zb, MSL)
    pltpu.sync_copy(matS.at[pl.ds(zb, MSL)], rel_h.at[pl.ds(ob, MSL)])
    plsc.subcore_barrier()


def _sc_body(rows_h, cols_h, a0_h, a1_h, ee1_h, ee2_h, x0_h, x1_h,
             emb1_h, emb2_h, zeros_h,
             rel_h, h_out_h,
             rA_v, cA_v, a0A_v, a1A_v, rB_v, cB_v, a0B_v, a1B_v,
             idx_v, upd_v, ee1_v, ee2_v, zbuf,
             xi0_v, xi1_v, g1_v, g2_v, matS, semi, semg, semz):
    cid = lax.axis_index("c")
    sid = lax.axis_index("s")

    wid = sid * 2 + cid
    nb = pl.multiple_of(wid * 32, 32)
    descs = [
        pltpu.async_copy(zeros_h, zbuf, semi),
        pltpu.async_copy(x0_h.at[pl.ds(nb, 32)], xi0_v, semi),
        pltpu.async_copy(x1_h.at[pl.ds(nb, 32)], xi1_v, semi),
        pltpu.async_copy(ee1_h, ee1_v, semi),
        pltpu.async_copy(ee2_h, ee2_v, semi),
    ]

    def edge_loads(r, bufs):
        npersub = REL_N[r] // NSUB
        eb = pl.multiple_of(REL_OFF[r] + sid * npersub, 8)
        srcs = (rows_h, cols_h, a0_h, a1_h)
        return [pltpu.async_copy(s.at[pl.ds(eb, npersub)],
                                 b.at[pl.ds(0, npersub)], semi)
                for s, b in zip(srcs, bufs)]

    bufsA = (rA_v, cA_v, a0A_v, a1A_v)
    bufsB = (rB_v, cB_v, a0B_v, a1B_v)

    @pl.when(cid == 0)
    def _():
        for d in edge_loads(0, bufsA) + edge_loads(2, bufsB):
            d.wait()

    @pl.when(cid == 1)
    def _():
        for d in edge_loads(1, bufsA) + edge_loads(3, bufsB):
            d.wait()

    for d in descs:
        d.wait()

    dg1 = pltpu.async_copy(emb1_h.at[xi0_v], g1_v, semg)
    dg2 = pltpu.async_copy(emb2_h.at[xi1_v], g2_v, semg)
    dg1.wait()
    dg2.wait()
    for j in range(32):
        for q in range(8):
            sl = pl.ds(q * 16, 16)
            g1_v[j, sl] = g1_v[j, sl] + g2_v[j, sl]
    pltpu.sync_copy(g1_v, h_out_h.at[pl.ds(nb, 32)])

    @pl.when(cid == 0)
    def _():
        _build_relation(0, sid, bufsA, zbuf, rel_h,
                        idx_v, upd_v, ee1_v, ee2_v, matS, semz, None)
        _build_relation(2, sid, bufsB, zbuf, rel_h,
                        idx_v, upd_v, ee1_v, ee2_v, matS, semz,
                        (idx_v, upd_v, 2))

    @pl.when(cid == 1)
    def _():
        _build_relation(1, sid, bufsA, zbuf, rel_h,
                        idx_v, upd_v, ee1_v, ee2_v, matS, semz, None)
        _build_relation(3, sid, bufsB, zbuf, rel_h,
                        idx_v, upd_v, ee1_v, ee2_v, matS, semz,
                        (idx_v, upd_v, 1))


@jax.jit
def _sc_scatter(rows_p, cols_p, a0_p, a1_p, ee1p, ee2p, x0, x1,
                emb1, emb2, zeros16k):
    mesh = plsc.VectorSubcoreMesh(core_axis_name="c", subcore_axis_name="s")
    return pl.kernel(
        _sc_body,
        out_type=[jax.ShapeDtypeStruct((4 * MAT,), f32),
                  jax.ShapeDtypeStruct((N, EMB), f32)],
        mesh=mesh,
        scratch_types=[
            pltpu.VMEM((256,), i32),
            pltpu.VMEM((256,), i32),
            pltpu.VMEM((256,), i32),
            pltpu.VMEM((256,), i32),
            pltpu.VMEM((128,), i32),
            pltpu.VMEM((128,), i32),
            pltpu.VMEM((128,), i32),
            pltpu.VMEM((128,), i32),
            pltpu.VMEM((2, 128), i32),
            pltpu.VMEM((2, 128), f32),
            pltpu.VMEM((16,), f32),
            pltpu.VMEM((16,), f32),
            pltpu.VMEM((ZB,), f32),
            pltpu.VMEM((32,), i32),
            pltpu.VMEM((32,), i32),
            pltpu.VMEM((32, 128), f32),
            pltpu.VMEM((32, 128), f32),
            pltpu.VMEM_SHARED((MAT,), f32),
            pltpu.SemaphoreType.DMA,
            pltpu.SemaphoreType.DMA,
            pltpu.SemaphoreType.DMA,
        ],
        compiler_params=pltpu.CompilerParams(needs_layout_passes=False),
    )(rows_p, cols_p, a0_p, a1_p, ee1p, ee2p, x0, x1, emb1, emb2, zeros16k)


def _tc_body(fb_ref, rel_ref, h_ref, gcnW_ref, gcnb_ref, batch_ref,
             linW_ref, linb_ref, out_ref):
    c = pl.program_id(0)
    dotb = functools.partial(lax.dot, preferred_element_type=f32)

    iota_r = lax.broadcasted_iota(i32, (N, N), 0)
    iota_c = lax.broadcasted_iota(i32, (N, N), 1)
    diag = iota_r == iota_c

    acc = [None] * 3
    for j in range(4):
        a = rel_ref[j].astype(f32)
        for m in range(3):
            term = fb_ref[c, m, j] * a
            acc[m] = term if j == 0 else acc[m] + term
    Am, Bm, Cm = [jnp.where(diag, acc[m] + fb_ref[c, m, 4],
                            acc[m]).astype(bf16) for m in range(3)]

    hWb = dotb(h_ref[...].astype(bf16), gcnW_ref[...].astype(bf16))
    H0 = dotb(Am, Bm)
    deg0 = jnp.sum(H0, axis=1, keepdims=True)
    inv0 = jnp.where(deg0 != 0.0, 1.0 / deg0, 0.0)
    H1 = dotb((inv0 * H0).astype(bf16), Cm)
    deg1 = jnp.sum(H1, axis=1, keepdims=True)
    inv1 = jnp.where(deg1 != 0.0, 1.0 / deg1, 0.0)
    Xc = jnp.maximum(dotb((inv1 * H1).astype(bf16), hWb.astype(bf16))
                     + gcnb_ref[...], 0.0)

    onehot = (lax.broadcasted_iota(i32, (NG, N), 0)
              == batch_ref[...]).astype(f32)
    counts = jnp.maximum(jnp.sum(onehot, axis=1, keepdims=True), 1.0)
    yc = lax.dot(onehot, Xc, precision=_HI,
                 preferred_element_type=f32) / counts
    contrib = dotb(yc.astype(bf16), linW_ref[0].astype(bf16))

    @pl.when(c == 0)
    def _():
        out_ref[...] = contrib + linb_ref[...]

    @pl.when(c != 0)
    def _():
        out_ref[...] = out_ref[...] + contrib


@jax.jit
def _tc_replica(fb, relstack, h, gcn_W, gcn_b2, batch2, lin_W3, lin_b2):
    return pl.pallas_call(
        _tc_body,
        grid=(NCH,),
        in_specs=[
            pl.BlockSpec(memory_space=pltpu.MemorySpace.SMEM),
            pl.BlockSpec((4, N, N), lambda c: (0, 0, 0)),
            pl.BlockSpec((N, EMB), lambda c: (0, 0)),
            pl.BlockSpec((EMB, WOUT), lambda c: (0, 0)),
            pl.BlockSpec((1, WOUT), lambda c: (0, 0)),
            pl.BlockSpec((1, N), lambda c: (0, 0)),
            pl.BlockSpec((1, WOUT, WOUT), lambda c: (c, 0, 0)),
            pl.BlockSpec((1, WOUT), lambda c: (0, 0)),
        ],
        out_specs=pl.BlockSpec((NG, WOUT), lambda c: (0, 0)),
        out_shape=jax.ShapeDtypeStruct((NG, WOUT), f32),
        compiler_params=pltpu.CompilerParams(
            dimension_semantics=("arbitrary",),
            vmem_limit_bytes=100 * 1024 * 1024),
    )(fb, relstack, h, gcn_W, gcn_b2, batch2, lin_W3, lin_b2)


def kernel(x, batch, single_edge_index, single_edge_attr, double_edge_index,
           double_edge_attr, triple_edge_index, triple_edge_attr,
           aromatic_edge_index, aromatic_edge_attr, x_embedding1, x_embedding2,
           edge_embedding1, edge_embedding2, l0_w1, l0_w2, l1_w1, gcn_W,
           gcn_b, lin_W, lin_b):
    idxs = (single_edge_index, double_edge_index, triple_edge_index,
            aromatic_edge_index)
    attrs = (single_edge_attr, double_edge_attr, triple_edge_attr,
             aromatic_edge_attr)

    rows_p = jnp.concatenate([i[0] for i in idxs]).astype(i32)
    cols_p = jnp.concatenate([i[1] for i in idxs]).astype(i32)
    a0_p = jnp.concatenate([a[:, 0] for a in attrs]).astype(i32)
    a1_p = jnp.concatenate([a[:, 1] for a in attrs]).astype(i32)

    ee1p = jnp.pad(edge_embedding1[:, 0].astype(f32), (0, 11))
    ee2p = jnp.pad(edge_embedding2[:, 0].astype(f32), (0, 13))

    x0 = x[:, 0].astype(i32)
    x1 = x[:, 1].astype(i32)
    zeros16k = jnp.zeros((ZB,), f32)

    relflat, h = _sc_scatter(rows_p, cols_p, a0_p, a1_p, ee1p, ee2p,
                             x0, x1, x_embedding1.astype(f32),
                             x_embedding2.astype(f32), zeros16k)
    relstack = relflat.reshape(4, N, N).astype(bf16)

    fb = jnp.stack([jax.nn.softmax(w.astype(f32), axis=1)
                    for w in (l0_w1, l0_w2, l1_w1)])
    fb = fb.astype(bf16).astype(f32).transpose(1, 0, 2)

    return _tc_replica(fb, relstack, h, gcn_W.astype(f32),
                       gcn_b.astype(f32).reshape(1, WOUT),
                       batch.astype(i32).reshape(1, N),
                       lin_W.astype(f32).reshape(NCH, WOUT, WOUT),
                       lin_b.astype(f32).reshape(1, WOUT))
